# baseline (speedup 1.0000x reference)
import jax
import jax.numpy as jnp
from jax import lax
from jax.experimental import pallas as pl
from jax.experimental.pallas import tpu as pltpu
import jax.experimental.pallas.tpu_sc as plsc

_NUM_CORES = 2
_NUM_SUBCORES = 16
_SUB = 1024


def _tc_argmax_kernel(x_ref, cb_ref, idx_ref, en_ref):
    i = pl.program_id(0)

    @pl.when(i == 0)
    def _prep():
        cb = cb_ref[...]
        cs = jnp.sum(cb * cb, axis=1, keepdims=True)
        en_ref[...] = cb / jnp.maximum(jnp.sqrt(cs), 1e-12)

    x = x_ref[...]
    xs = jnp.sum(x * x, axis=1, keepdims=True)
    xn = x / jnp.maximum(jnp.sqrt(xs), 1e-12)

    dot = jax.lax.dot_general(
        xn, en_ref[...], (((1,), (1,)), ((), ())),
        preferred_element_type=jnp.float32)
    idx_ref[0, 0, :] = jnp.argmax(dot, axis=1).astype(jnp.int32)


def _sc_gather_loss_kernel(idx_hbm, cb_hbm, x_hbm, q_hbm, part_hbm,
                           idx_v, rows_v, x_v, acc_v,
                           sem_i, sem_g, sem_x, sem_o, sem_p):
    c = lax.axis_index("c")
    s = lax.axis_index("s")
    tile = c * _NUM_SUBCORES + s
    n_tiles = _NUM_CORES * _NUM_SUBCORES
    n = q_hbm.shape[0]
    per_tile = n // n_tiles
    n_sub = per_tile // _SUB

    acc_v[...] = jnp.zeros((16,), jnp.float32)

    def sub_chunk(j, _):
        base = tile * per_tile + j * _SUB
        cp_i = pltpu.make_async_copy(
            idx_hbm.at[pl.ds(base, _SUB)], idx_v, sem_i)
        cp_i.start()
        cp_x = pltpu.make_async_copy(
            x_hbm.at[pl.ds(base, _SUB)], x_v, sem_x)
        cp_x.start()
        cp_i.wait()
        cp_g = pltpu.make_async_copy(cb_hbm.at[idx_v], rows_v, sem_g)
        cp_g.start()
        cp_g.wait()
        cp_o = pltpu.make_async_copy(
            rows_v, q_hbm.at[pl.ds(base, _SUB)], sem_o)
        cp_o.start()
        cp_x.wait()

        def tok(i, accs):
            a0, a1 = accs
            d0 = rows_v[i, pl.ds(0, 16)] - x_v[i, pl.ds(0, 16)]
            d1 = rows_v[i, pl.ds(16, 16)] - x_v[i, pl.ds(16, 16)]
            return a0 + d0 * d0, a1 + d1 * d1

        z = jnp.zeros((16,), jnp.float32)
        a0, a1 = lax.fori_loop(0, _SUB, tok, (z, z))
        acc_v[...] += a0 + a1
        cp_o.wait()
        return 0

    lax.fori_loop(0, n_sub, sub_chunk, 0)

    cp_p = pltpu.make_async_copy(acc_v, part_hbm.at[tile], sem_p)
    cp_p.start()
    cp_p.wait()


def kernel(x, codebook):
    b, t, d = x.shape
    k = codebook.shape[0]
    n = b * t
    blk = 1024
    chunks = 2
    nc = n // chunks
    nbc = nc // blk
    xf = x.reshape(n, d)

    n_tiles = _NUM_CORES * _NUM_SUBCORES
    sc_gather = pl.kernel(
        _sc_gather_loss_kernel,
        out_type=[
            jax.ShapeDtypeStruct((nc, d), jnp.float32),
            jax.ShapeDtypeStruct((n_tiles, 16), jnp.float32),
        ],
        mesh=plsc.VectorSubcoreMesh(
            core_axis_name="c", subcore_axis_name="s",
            num_cores=_NUM_CORES, num_subcores=_NUM_SUBCORES),
        compiler_params=pltpu.CompilerParams(use_tc_tiling_on_sc=False),
        scratch_types=[
            pltpu.VMEM((_SUB,), jnp.int32),
            pltpu.VMEM((_SUB, d), jnp.float32),
            pltpu.VMEM((_SUB, d), jnp.float32),
            pltpu.VMEM((16,), jnp.float32),
            pltpu.SemaphoreType.DMA,
            pltpu.SemaphoreType.DMA,
            pltpu.SemaphoreType.DMA,
            pltpu.SemaphoreType.DMA,
            pltpu.SemaphoreType.DMA,
        ],
    )

    idx_cs, q_cs, part_cs = [], [], []
    for ci in range(chunks):
        xc = lax.slice_in_dim(xf, ci * nc, (ci + 1) * nc, axis=0)
        idx_c = pl.pallas_call(
            _tc_argmax_kernel,
            grid=(nbc,),
            in_specs=[
                pl.BlockSpec((blk, d), lambda i: (i, 0)),
                pl.BlockSpec((k, d), lambda i: (0, 0)),
            ],
            out_specs=pl.BlockSpec((1, 1, blk), lambda i: (i, 0, 0)),
            out_shape=jax.ShapeDtypeStruct((nbc, 1, blk), jnp.int32),
            scratch_shapes=[pltpu.VMEM((k, d), jnp.float32)],
        )(xc, codebook)
        q_c, part_c = sc_gather(idx_c.reshape(nc), codebook, xc)
        idx_cs.append(idx_c)
        q_cs.append(q_c)
        part_cs.append(part_c)

    q = jnp.concatenate(q_cs, axis=0)
    indices = jnp.concatenate(
        [i.reshape(nc) for i in idx_cs], axis=0).reshape(b, t)
    quantized_st = q.reshape(b, t, d)
    quantize_loss = (1.25 / (n * d)) * sum(jnp.sum(p) for p in part_cs)
    return (quantized_st, indices, quantize_loss)


# speedup vs baseline: 1.1095x; 1.1095x over previous
import jax
import jax.numpy as jnp
from jax import lax
from jax.experimental import pallas as pl
from jax.experimental.pallas import tpu as pltpu
import jax.experimental.pallas.tpu_sc as plsc

_NUM_CORES = 2
_NUM_SUBCORES = 16
_SUB = 1024


def _tc_argmax_kernel(x_ref, cb_ref, idx_ref, en_ref):
    i = pl.program_id(0)

    @pl.when(i == 0)
    def _prep():
        cb = cb_ref[...]
        cs = jnp.sum(cb * cb, axis=1, keepdims=True)
        en_ref[...] = cb / jnp.maximum(jnp.sqrt(cs), 1e-12)

    en = en_ref[...]
    h = x_ref.shape[0] // 2

    def _half(x):
        xs = jnp.sum(x * x, axis=1, keepdims=True)
        xn = x / jnp.maximum(jnp.sqrt(xs), 1e-12)
        dot = jax.lax.dot_general(
            xn, en, (((1,), (1,)), ((), ())),
            preferred_element_type=jnp.float32)
        return jnp.argmax(dot, axis=1).astype(jnp.int32)

    idx_ref[0, 0, :h] = _half(x_ref[:h, :])
    idx_ref[0, 0, h:] = _half(x_ref[h:, :])


def _sc_gather_loss_kernel(idx_hbm, cb_hbm, x_hbm, q_hbm, part_hbm,
                           idx_v, rows_v, x_v, acc_v,
                           sem_i, sem_g, sem_x, sem_o, sem_p):
    c = lax.axis_index("c")
    s = lax.axis_index("s")
    tile = c * _NUM_SUBCORES + s
    n_tiles = _NUM_CORES * _NUM_SUBCORES
    n = q_hbm.shape[0]
    per_tile = n // n_tiles
    n_sub = per_tile // _SUB

    acc_v[...] = jnp.zeros((16,), jnp.float32)

    def sub_chunk(j, _):
        base = tile * per_tile + j * _SUB
        cp_i = pltpu.make_async_copy(
            idx_hbm.at[pl.ds(base, _SUB)], idx_v, sem_i)
        cp_i.start()
        cp_x = pltpu.make_async_copy(
            x_hbm.at[pl.ds(base, _SUB)], x_v, sem_x)
        cp_x.start()
        cp_i.wait()
        cp_g = pltpu.make_async_copy(cb_hbm.at[idx_v], rows_v, sem_g)
        cp_g.start()
        cp_g.wait()
        cp_o = pltpu.make_async_copy(
            rows_v, q_hbm.at[pl.ds(base, _SUB)], sem_o)
        cp_o.start()
        cp_x.wait()

        def tok(i, accs):
            a0, a1 = accs
            d0 = rows_v[i, pl.ds(0, 16)] - x_v[i, pl.ds(0, 16)]
            d1 = rows_v[i, pl.ds(16, 16)] - x_v[i, pl.ds(16, 16)]
            return a0 + d0 * d0, a1 + d1 * d1

        z = jnp.zeros((16,), jnp.float32)
        a0, a1 = lax.fori_loop(0, _SUB, tok, (z, z))
        acc_v[...] += a0 + a1
        cp_o.wait()
        return 0

    lax.fori_loop(0, n_sub, sub_chunk, 0)

    cp_p = pltpu.make_async_copy(acc_v, part_hbm.at[tile], sem_p)
    cp_p.start()
    cp_p.wait()


def kernel(x, codebook):
    b, t, d = x.shape
    k = codebook.shape[0]
    n = b * t
    blk = 1024
    nb = n // blk
    xf = x.reshape(n, d)

    idx = pl.pallas_call(
        _tc_argmax_kernel,
        grid=(nb,),
        in_specs=[
            pl.BlockSpec((blk, d), lambda i: (i, 0)),
            pl.BlockSpec((k, d), lambda i: (0, 0)),
        ],
        out_specs=pl.BlockSpec((1, 1, blk), lambda i: (i, 0, 0)),
        out_shape=jax.ShapeDtypeStruct((nb, 1, blk), jnp.int32),
        scratch_shapes=[pltpu.VMEM((k, d), jnp.float32)],
    )(xf, codebook)
    idx_flat = idx.reshape(n)

    n_tiles = _NUM_CORES * _NUM_SUBCORES
    sc_gather = pl.kernel(
        _sc_gather_loss_kernel,
        out_type=[
            jax.ShapeDtypeStruct((n, d), jnp.float32),
            jax.ShapeDtypeStruct((n_tiles, 16), jnp.float32),
        ],
        mesh=plsc.VectorSubcoreMesh(
            core_axis_name="c", subcore_axis_name="s",
            num_cores=_NUM_CORES, num_subcores=_NUM_SUBCORES),
        compiler_params=pltpu.CompilerParams(use_tc_tiling_on_sc=False),
        scratch_types=[
            pltpu.VMEM((_SUB,), jnp.int32),
            pltpu.VMEM((_SUB, d), jnp.float32),
            pltpu.VMEM((_SUB, d), jnp.float32),
            pltpu.VMEM((16,), jnp.float32),
            pltpu.SemaphoreType.DMA,
            pltpu.SemaphoreType.DMA,
            pltpu.SemaphoreType.DMA,
            pltpu.SemaphoreType.DMA,
            pltpu.SemaphoreType.DMA,
        ],
    )
    q, parts = sc_gather(idx_flat, codebook, xf)

    quantized_st = q.reshape(b, t, d)
    indices = idx.reshape(b, t)
    quantize_loss = (1.25 / (n * d)) * jnp.sum(parts)
    return (quantized_st, indices, quantize_loss)


# speedup vs baseline: 1.1678x; 1.0526x over previous
import jax
import jax.numpy as jnp
from jax import lax
from jax.experimental import pallas as pl
from jax.experimental.pallas import tpu as pltpu
import jax.experimental.pallas.tpu_sc as plsc

_NUM_CORES = 2
_NUM_SUBCORES = 16
_SUB = 1024


def _tc_argmax_kernel(x_ref, cb_ref, idx_ref, en_ref):
    i = pl.program_id(0)

    @pl.when(i == 0)
    def _prep():
        cb = cb_ref[...]
        cs = jnp.sum(cb * cb, axis=1, keepdims=True)
        en_ref[...] = cb / jnp.maximum(jnp.sqrt(cs), 1e-12)

    x = x_ref[...]
    xs = jnp.sum(x * x, axis=1, keepdims=True)
    xn = x / jnp.maximum(jnp.sqrt(xs), 1e-12)

    dot = jax.lax.dot_general(
        xn, en_ref[...], (((1,), (1,)), ((), ())),
        preferred_element_type=jnp.float32)
    idx_ref[0, 0, :] = jnp.argmax(dot, axis=1).astype(jnp.int32)


def _sc_gather_loss_kernel(idx_hbm, cb_hbm, x_hbm, q_hbm, part_hbm,
                           idx_v, rows_v, x_v, acc_v,
                           sem_i, sem_g, sem_x, sem_o, sem_p):
    c = lax.axis_index("c")
    s = lax.axis_index("s")
    tile = c * _NUM_SUBCORES + s
    n_tiles = _NUM_CORES * _NUM_SUBCORES
    n = q_hbm.shape[0]
    per_tile = n // n_tiles
    n_sub = per_tile // _SUB

    acc_v[...] = jnp.zeros((16,), jnp.float32)

    def sub_chunk(j, _):
        base = tile * per_tile + j * _SUB
        cp_i = pltpu.make_async_copy(
            idx_hbm.at[pl.ds(base, _SUB)], idx_v, sem_i)
        cp_i.start()
        cp_x = pltpu.make_async_copy(
            x_hbm.at[pl.ds(base, _SUB)], x_v, sem_x)
        cp_x.start()
        cp_i.wait()
        cp_g = pltpu.make_async_copy(cb_hbm.at[idx_v], rows_v, sem_g)
        cp_g.start()
        cp_g.wait()
        cp_o = pltpu.make_async_copy(
            rows_v, q_hbm.at[pl.ds(base, _SUB)], sem_o)
        cp_o.start()
        cp_x.wait()

        def tok4(i, accs):
            a0, a1 = accs
            for u in range(4):
                r = i * 4 + u
                d0 = rows_v[r, pl.ds(0, 16)] - x_v[r, pl.ds(0, 16)]
                d1 = rows_v[r, pl.ds(16, 16)] - x_v[r, pl.ds(16, 16)]
                a0 += d0 * d0
                a1 += d1 * d1
            return a0, a1

        z = jnp.zeros((16,), jnp.float32)
        a0, a1 = lax.fori_loop(0, _SUB // 4, tok4, (z, z))
        acc_v[...] += a0 + a1
        cp_o.wait()
        return 0

    lax.fori_loop(0, n_sub, sub_chunk, 0)

    cp_p = pltpu.make_async_copy(acc_v, part_hbm.at[tile], sem_p)
    cp_p.start()
    cp_p.wait()


def kernel(x, codebook):
    b, t, d = x.shape
    k = codebook.shape[0]
    n = b * t
    blk = 1024
    nb = n // blk
    xf = x.reshape(n, d)

    idx = pl.pallas_call(
        _tc_argmax_kernel,
        grid=(nb,),
        in_specs=[
            pl.BlockSpec((blk, d), lambda i: (i, 0)),
            pl.BlockSpec((k, d), lambda i: (0, 0)),
        ],
        out_specs=pl.BlockSpec((1, 1, blk), lambda i: (i, 0, 0)),
        out_shape=jax.ShapeDtypeStruct((nb, 1, blk), jnp.int32),
        scratch_shapes=[pltpu.VMEM((k, d), jnp.float32)],
    )(xf, codebook)
    idx_flat = idx.reshape(n)

    n_tiles = _NUM_CORES * _NUM_SUBCORES
    sc_gather = pl.kernel(
        _sc_gather_loss_kernel,
        out_type=[
            jax.ShapeDtypeStruct((n, d), jnp.float32),
            jax.ShapeDtypeStruct((n_tiles, 16), jnp.float32),
        ],
        mesh=plsc.VectorSubcoreMesh(
            core_axis_name="c", subcore_axis_name="s",
            num_cores=_NUM_CORES, num_subcores=_NUM_SUBCORES),
        compiler_params=pltpu.CompilerParams(use_tc_tiling_on_sc=False),
        scratch_types=[
            pltpu.VMEM((_SUB,), jnp.int32),
            pltpu.VMEM((_SUB, d), jnp.float32),
            pltpu.VMEM((_SUB, d), jnp.float32),
            pltpu.VMEM((16,), jnp.float32),
            pltpu.SemaphoreType.DMA,
            pltpu.SemaphoreType.DMA,
            pltpu.SemaphoreType.DMA,
            pltpu.SemaphoreType.DMA,
            pltpu.SemaphoreType.DMA,
        ],
    )
    q, parts = sc_gather(idx_flat, codebook, xf)

    quantized_st = q.reshape(b, t, d)
    indices = idx.reshape(b, t)
    quantize_loss = (1.25 / (n * d)) * jnp.sum(parts)
    return (quantized_st, indices, quantize_loss)


# speedup vs baseline: 1.1746x; 1.0058x over previous
import jax
import jax.numpy as jnp
from jax import lax
from jax.experimental import pallas as pl
from jax.experimental.pallas import tpu as pltpu
import jax.experimental.pallas.tpu_sc as plsc

_NUM_CORES = 2
_NUM_SUBCORES = 16
_SUB = 512


def _tc_argmax_kernel(x_ref, cb_ref, idx_ref, en_ref):
    i = pl.program_id(0)

    @pl.when(i == 0)
    def _prep():
        cb = cb_ref[...]
        cs = jnp.sum(cb * cb, axis=1, keepdims=True)
        en_ref[...] = cb / jnp.maximum(jnp.sqrt(cs), 1e-12)

    x = x_ref[...]
    xs = jnp.sum(x * x, axis=1, keepdims=True)
    xn = x / jnp.maximum(jnp.sqrt(xs), 1e-12)

    dot = jax.lax.dot_general(
        xn, en_ref[...], (((1,), (1,)), ((), ())),
        preferred_element_type=jnp.float32)
    idx_ref[0, 0, :] = jnp.argmax(dot, axis=1).astype(jnp.int32)


def _sc_gather_loss_kernel(idx_hbm, cb_hbm, x_hbm, q_hbm, part_hbm,
                           idx_v, x_v, rows_a, rows_b, acc_v,
                           sem_i, sem_x, sem_ga, sem_gb, sem_oa, sem_ob,
                           sem_p):
    c = lax.axis_index("c")
    s = lax.axis_index("s")
    tile = c * _NUM_SUBCORES + s
    n_tiles = _NUM_CORES * _NUM_SUBCORES
    n = q_hbm.shape[0]
    per_tile = n // n_tiles
    n_sub = per_tile // _SUB
    tbase = tile * per_tile

    cp_i = pltpu.make_async_copy(
        idx_hbm.at[pl.ds(tbase, per_tile)], idx_v, sem_i)
    cp_i.start()
    cp_x = pltpu.make_async_copy(
        x_hbm.at[pl.ds(tbase, per_tile)], x_v, sem_x)
    cp_x.start()
    cp_i.wait()

    rows = [rows_a, rows_b]
    sem_g = [sem_ga, sem_gb]
    sem_o = [sem_oa, sem_ob]
    gathers = [None, None]
    outs = [None, None]

    def _gather(j):
        buf = j % 2
        g = pltpu.make_async_copy(
            cb_hbm.at[idx_v.at[pl.ds(j * _SUB, _SUB)]], rows[buf],
            sem_g[buf])
        g.start()
        gathers[buf] = g

    _gather(0)
    cp_x.wait()
    a0 = jnp.zeros((16,), jnp.float32)
    a1 = jnp.zeros((16,), jnp.float32)

    for j in range(n_sub):
        cur = j % 2
        nxt = (j + 1) % 2
        gathers[cur].wait()
        if j + 1 < n_sub:
            if outs[nxt] is not None:
                outs[nxt].wait()
            _gather(j + 1)
        o = pltpu.make_async_copy(
            rows[cur], q_hbm.at[pl.ds(tbase + j * _SUB, _SUB)], sem_o[cur])
        o.start()
        outs[cur] = o

        def tok4(i, accs, _j=j, _cur=cur):
            b0, b1 = accs
            for u in range(4):
                r = i * 4 + u
                xr = _j * _SUB + r
                d0 = rows[_cur][r, pl.ds(0, 16)] - x_v[xr, pl.ds(0, 16)]
                d1 = rows[_cur][r, pl.ds(16, 16)] - x_v[xr, pl.ds(16, 16)]
                b0 += d0 * d0
                b1 += d1 * d1
            return b0, b1

        a0, a1 = lax.fori_loop(0, _SUB // 4, tok4, (a0, a1))

    acc_v[...] = a0 + a1
    for o in outs:
        if o is not None:
            o.wait()

    cp_p = pltpu.make_async_copy(acc_v, part_hbm.at[tile], sem_p)
    cp_p.start()
    cp_p.wait()


def kernel(x, codebook):
    b, t, d = x.shape
    k = codebook.shape[0]
    n = b * t
    blk = 1024
    nb = n // blk
    xf = x.reshape(n, d)

    idx = pl.pallas_call(
        _tc_argmax_kernel,
        grid=(nb,),
        in_specs=[
            pl.BlockSpec((blk, d), lambda i: (i, 0)),
            pl.BlockSpec((k, d), lambda i: (0, 0)),
        ],
        out_specs=pl.BlockSpec((1, 1, blk), lambda i: (i, 0, 0)),
        out_shape=jax.ShapeDtypeStruct((nb, 1, blk), jnp.int32),
        scratch_shapes=[pltpu.VMEM((k, d), jnp.float32)],
    )(xf, codebook)
    idx_flat = idx.reshape(n)

    n_tiles = _NUM_CORES * _NUM_SUBCORES
    sc_gather = pl.kernel(
        _sc_gather_loss_kernel,
        out_type=[
            jax.ShapeDtypeStruct((n, d), jnp.float32),
            jax.ShapeDtypeStruct((n_tiles, 16), jnp.float32),
        ],
        mesh=plsc.VectorSubcoreMesh(
            core_axis_name="c", subcore_axis_name="s",
            num_cores=_NUM_CORES, num_subcores=_NUM_SUBCORES),
        compiler_params=pltpu.CompilerParams(use_tc_tiling_on_sc=False),
        scratch_types=[
            pltpu.VMEM((n // n_tiles,), jnp.int32),
            pltpu.VMEM((n // n_tiles, d), jnp.float32),
            pltpu.VMEM((_SUB, d), jnp.float32),
            pltpu.VMEM((_SUB, d), jnp.float32),
            pltpu.VMEM((16,), jnp.float32),
            pltpu.SemaphoreType.DMA,
            pltpu.SemaphoreType.DMA,
            pltpu.SemaphoreType.DMA,
            pltpu.SemaphoreType.DMA,
            pltpu.SemaphoreType.DMA,
            pltpu.SemaphoreType.DMA,
            pltpu.SemaphoreType.DMA,
        ],
    )
    q, parts = sc_gather(idx_flat, codebook, xf)

    quantized_st = q.reshape(b, t, d)
    indices = idx.reshape(b, t)
    quantize_loss = (1.25 / (n * d)) * jnp.sum(parts)
    return (quantized_st, indices, quantize_loss)
